# Initial kernel scaffold; baseline (speedup 1.0000x reference)
import functools

import jax
import jax.numpy as jnp
from jax import lax
from jax.experimental import pallas as pl
from jax.experimental.pallas import tpu as pltpu
from jax.experimental.pallas import tpu_sc as plsc

B, N, D = 32, 8192, 128
G = 819
L = 16
NV = N // L
RADIX_BITS = 11
NBINS = 1 << RADIX_BITS
NT = -(-G // L) + (0 if G % L == 0 else 1)


def _maxkey_body(s_ref, key_ref):
    v = jnp.max(s_ref[...], axis=-1) + 0.0
    b = lax.bitcast_convert_type(v, jnp.int32)
    key_ref[...] = jnp.where(b < 0, b, ~b & jnp.int32(0x7FFFFFFF))


def _maxkey(s):
    nj = 4
    return pl.pallas_call(
        _maxkey_body,
        grid=(B, nj),
        in_specs=[pl.BlockSpec((1, N // nj, D), lambda b, j: (b, j, 0))],
        out_specs=pl.BlockSpec((1, N // nj), lambda b, j: (b, j)),
        out_shape=jax.ShapeDtypeStruct((B, N), jnp.int32),
    )(s)


def _digit(k, shift):
    return jnp.bitwise_and(lax.shift_right_logical(k, shift), NBINS - 1)


def _recover(k):
    b = jnp.where(k >= 0, ~k & jnp.int32(0x7FFFFFFF), k)
    return plsc.bitcast(b, jnp.float32)


def _radix_pass(shift, src_k, src_i, dst_k, dst_i, hist):
    def zero(j, c):
        hist[pl.ds(j * L, L)] = jnp.zeros((L,), jnp.int32)
        return c

    lax.fori_loop(0, NBINS // L, zero, 0, unroll=4)

    def histo(j, c):
        k = src_k[pl.ds(j * L, L)]
        d = _digit(k, shift)
        cnt, last = plsc.scan_count(d)
        plsc.addupdate_scatter(hist, [d], cnt + 1, mask=last)
        return c

    lax.fori_loop(0, NV, histo, 0, unroll=4)

    def psum(j, carry):
        h = hist[pl.ds(j * L, L)]
        inc = plsc.cumsum(h)
        hist[pl.ds(j * L, L)] = inc - h + carry
        return carry + jnp.max(inc)

    lax.fori_loop(0, NBINS // L, psum, jnp.int32(0))

    def perm(j, c):
        k = src_k[pl.ds(j * L, L)]
        i = src_i[pl.ds(j * L, L)]
        d = _digit(k, shift)
        cnt, last = plsc.scan_count(d)
        pos = plsc.load_gather(hist, [d]) + cnt
        plsc.store_scatter(dst_k, [pos], k)
        plsc.store_scatter(dst_i, [pos], i)
        plsc.addupdate_scatter(hist, [d], cnt + 1, mask=last)
        return c

    lax.fori_loop(0, NV, perm, 0, unroll=2)


def _sc_body(keys_hbm, pw_hbm, idx_hbm, ka, kb, ia, ib, hist, w):
    nc = 2
    wid = lax.axis_index("s") * nc + lax.axis_index("c")
    pltpu.sync_copy(keys_hbm.at[wid], ka)

    lane = lax.iota(jnp.int32, L)

    def init(j, c):
        ia[pl.ds(j * L, L)] = lane + j * L
        w[pl.ds(j * L, L)] = jnp.zeros((L,), jnp.float32)
        return c

    lax.fori_loop(0, NV, init, 0, unroll=4)

    _radix_pass(0, ka, ia, kb, ib, hist)
    _radix_pass(RADIX_BITS, kb, ib, ka, ia, hist)
    _radix_pass(2 * RADIX_BITS, ka, ia, kb, ib, hist)

    vmax = jnp.max(_recover(kb[pl.ds(0, L)]))

    def tsum(j, acc):
        v = _recover(kb[pl.ds(j * L, L)])
        m = lane + j * L < G
        return acc + jnp.where(m, jnp.exp(v - vmax), 0.0)

    acc = lax.fori_loop(0, NT, tsum, jnp.zeros((L,), jnp.float32), unroll=2)
    tsumv = jnp.sum(acc)

    def tscat(j, c):
        v = _recover(kb[pl.ds(j * L, L)])
        i = ib[pl.ds(j * L, L)]
        m = lane + j * L < G
        plsc.store_scatter(w, [i], jnp.exp(v - vmax) / tsumv, mask=m)
        return c

    lax.fori_loop(0, NT, tscat, 0, unroll=2)

    base = NV - NT
    vmin = jnp.min(_recover(kb[pl.ds((NV - 1) * L, L)]))

    def bsum(j, acc):
        v = _recover(kb[pl.ds((base + j) * L, L)])
        m = lane + (base + j) * L >= N - G
        return acc + jnp.where(m, jnp.exp(vmin - v), 0.0)

    bacc = lax.fori_loop(0, NT, bsum, jnp.zeros((L,), jnp.float32), unroll=2)
    bsumv = jnp.sum(bacc)

    def bscat(j, c):
        v = _recover(kb[pl.ds((base + j) * L, L)])
        i = ib[pl.ds((base + j) * L, L)]
        m = lane + (base + j) * L >= N - G
        plsc.store_scatter(w, [i], -jnp.exp(vmin - v) / bsumv, mask=m)
        return c

    lax.fori_loop(0, NT, bscat, 0, unroll=2)

    pltpu.sync_copy(w, pw_hbm.at[wid])
    pltpu.sync_copy(ib, idx_hbm.at[wid])


@functools.cache
def _sc_sort():
    mesh = plsc.VectorSubcoreMesh(core_axis_name="c", subcore_axis_name="s")
    return pl.kernel(
        _sc_body,
        mesh=mesh,
        out_type=[
            jax.ShapeDtypeStruct((B, N), jnp.float32),
            jax.ShapeDtypeStruct((B, N), jnp.int32),
        ],
        scratch_types=[
            pltpu.VMEM((N,), jnp.int32),
            pltpu.VMEM((N,), jnp.int32),
            pltpu.VMEM((N,), jnp.int32),
            pltpu.VMEM((N,), jnp.int32),
            pltpu.VMEM((NBINS,), jnp.int32),
            pltpu.VMEM((N,), jnp.float32),
        ],
    )


def kernel(s):
    keys = _maxkey(s)
    pw, idx = _sc_sort()(keys)
    return pw, idx


# speedup vs baseline: 3.2565x; 3.2565x over previous
import functools

import jax
import jax.numpy as jnp
from jax import lax
from jax.experimental import pallas as pl
from jax.experimental.pallas import tpu as pltpu
from jax.experimental.pallas import tpu_sc as plsc

B, N, D = 32, 8192, 128
G = 819
L = 16
NV = N // L
RADIX_BITS = 11
NBINS = 1 << RADIX_BITS
NT = -(-G // L)


def _maxkey_body(s_ref, key_ref):
    v = jnp.max(s_ref[...], axis=-1) + 0.0
    b = lax.bitcast_convert_type(v, jnp.int32)
    key_ref[...] = jnp.where(b < 0, b, ~b & jnp.int32(0x7FFFFFFF))


def _maxkey(s):
    nb, nj = 8, 8
    return pl.pallas_call(
        _maxkey_body,
        grid=(B // nb, nj),
        in_specs=[pl.BlockSpec((nb, N // nj, D), lambda b, j: (b, j, 0))],
        out_specs=pl.BlockSpec((nb, N // nj), lambda b, j: (b, j)),
        out_shape=jax.ShapeDtypeStruct((B, N), jnp.int32),
    )(s)


def _digit(k, shift):
    return jnp.bitwise_and(lax.shift_right_logical(k, shift), NBINS - 1)


def _recover(k):
    b = jnp.where(k >= 0, ~k & jnp.int32(0x7FFFFFFF), k)
    return plsc.bitcast(b, jnp.float32)


def _radix_pass(shift, src_k, src_i, dst_k, dst_i, hist):
    def zero(j, c):
        hist[pl.ds(j * L, L)] = jnp.zeros((L,), jnp.int32)
        return c

    lax.fori_loop(0, NBINS // L, zero, 0, unroll=4)

    def histo(j, c):
        k = src_k[pl.ds(j * L, L)]
        d = _digit(k, shift)
        cnt, last = plsc.scan_count(d)
        plsc.addupdate_scatter(hist, [d], cnt, mask=last)
        return c

    lax.fori_loop(0, NV, histo, 0, unroll=4)

    def psum(j, carry):
        h = hist[pl.ds(j * L, L)]
        inc = plsc.cumsum(h)
        hist[pl.ds(j * L, L)] = inc - h + carry
        return carry + jnp.max(inc)

    lax.fori_loop(0, NBINS // L, psum, jnp.int32(0))

    def perm(j, c):
        k = src_k[pl.ds(j * L, L)]
        i = src_i[pl.ds(j * L, L)]
        d = _digit(k, shift)
        cnt, last = plsc.scan_count(d)
        pos = plsc.load_gather(hist, [d]) + cnt - 1
        plsc.store_scatter(dst_k, [pos], k)
        plsc.store_scatter(dst_i, [pos], i)
        plsc.addupdate_scatter(hist, [d], cnt, mask=last)
        return c

    lax.fori_loop(0, NV, perm, 0, unroll=2)


def _sc_body(keys_hbm, pw_hbm, idx_hbm, ka, kb, ia, ib, hist, w):
    nc = 2
    wid = lax.axis_index("s") * nc + lax.axis_index("c")
    pltpu.sync_copy(keys_hbm.at[wid], ka)

    lane = lax.iota(jnp.int32, L)

    def init(j, c):
        ia[pl.ds(j * L, L)] = lane + j * L
        w[pl.ds(j * L, L)] = jnp.zeros((L,), jnp.float32)
        return c

    lax.fori_loop(0, NV, init, 0, unroll=4)

    _radix_pass(0, ka, ia, kb, ib, hist)
    _radix_pass(RADIX_BITS, kb, ib, ka, ia, hist)
    _radix_pass(2 * RADIX_BITS, ka, ia, kb, ib, hist)

    vmax = jnp.max(_recover(kb[pl.ds(0, L)]))

    def tsum(j, acc):
        v = _recover(kb[pl.ds(j * L, L)])
        m = lane + j * L < G
        return acc + jnp.where(m, jnp.exp(v - vmax), 0.0)

    acc = lax.fori_loop(0, NT, tsum, jnp.zeros((L,), jnp.float32), unroll=2)
    tsumv = jnp.sum(acc)

    def tscat(j, c):
        v = _recover(kb[pl.ds(j * L, L)])
        i = ib[pl.ds(j * L, L)]
        m = lane + j * L < G
        plsc.store_scatter(w, [i], jnp.exp(v - vmax) / tsumv, mask=m)
        return c

    lax.fori_loop(0, NT, tscat, 0, unroll=2)

    base = NV - NT
    vmin = jnp.min(_recover(kb[pl.ds((NV - 1) * L, L)]))

    def bsum(j, acc):
        v = _recover(kb[pl.ds((base + j) * L, L)])
        m = lane + (base + j) * L >= N - G
        return acc + jnp.where(m, jnp.exp(vmin - v), 0.0)

    bacc = lax.fori_loop(0, NT, bsum, jnp.zeros((L,), jnp.float32), unroll=2)
    bsumv = jnp.sum(bacc)

    def bscat(j, c):
        v = _recover(kb[pl.ds((base + j) * L, L)])
        i = ib[pl.ds((base + j) * L, L)]
        m = lane + (base + j) * L >= N - G
        plsc.store_scatter(w, [i], -jnp.exp(vmin - v) / bsumv, mask=m)
        return c

    lax.fori_loop(0, NT, bscat, 0, unroll=2)

    pltpu.sync_copy(w, pw_hbm.at[wid])
    pltpu.sync_copy(ib, idx_hbm.at[wid])


@functools.cache
def _sc_sort():
    mesh = plsc.VectorSubcoreMesh(core_axis_name="c", subcore_axis_name="s")
    return pl.kernel(
        _sc_body,
        mesh=mesh,
        compiler_params=pltpu.CompilerParams(needs_layout_passes=False),
        out_type=[
            jax.ShapeDtypeStruct((B, N), jnp.float32),
            jax.ShapeDtypeStruct((B, N), jnp.int32),
        ],
        scratch_types=[
            pltpu.VMEM((N,), jnp.int32),
            pltpu.VMEM((N,), jnp.int32),
            pltpu.VMEM((N,), jnp.int32),
            pltpu.VMEM((N,), jnp.int32),
            pltpu.VMEM((NBINS,), jnp.int32),
            pltpu.VMEM((N,), jnp.float32),
        ],
    )


def kernel(s):
    keys = _maxkey(s)
    pw, idx = _sc_sort()(keys)
    return pw, idx


# speedup vs baseline: 3.5003x; 1.0749x over previous
import functools

import jax
import jax.numpy as jnp
from jax import lax
from jax.experimental import pallas as pl
from jax.experimental.pallas import tpu as pltpu
from jax.experimental.pallas import tpu_sc as plsc

B, N, D = 32, 8192, 128
G = 819
L = 16
NV = N // L
RADIX_BITS = 11
NBINS = 1 << RADIX_BITS
NT = -(-G // L)


def _maxkey_body(s_ref, key_ref):
    v = jnp.max(s_ref[...], axis=-1) + 0.0
    b = lax.bitcast_convert_type(v, jnp.int32)
    key_ref[...] = jnp.where(b < 0, b, ~b & jnp.int32(0x7FFFFFFF))


def _maxkey(s):
    nb, nj = 8, 8
    return pl.pallas_call(
        _maxkey_body,
        grid=(B // nb, nj),
        in_specs=[pl.BlockSpec((nb, N // nj, D), lambda b, j: (b, j, 0))],
        out_specs=pl.BlockSpec((nb, N // nj), lambda b, j: (b, j)),
        out_shape=jax.ShapeDtypeStruct((B, N), jnp.int32),
    )(s)


def _digit(k, shift):
    return jnp.bitwise_and(lax.shift_right_logical(k, shift), NBINS - 1)


def _recover(k):
    b = jnp.where(k >= 0, ~k & jnp.int32(0x7FFFFFFF), k)
    return plsc.bitcast(b, jnp.float32)


def _psum(hist):
    def psum(j, carry):
        h = hist[pl.ds(j * L, L)]
        inc = plsc.cumsum(h)
        hist[pl.ds(j * L, L)] = inc - h + carry
        return carry + jnp.max(inc)

    lax.fori_loop(0, NBINS // L, psum, jnp.int32(0))


def _perm_pass(shift, src_k, src_i, dst_k, dst_i, hist, nxt):
    nshift = shift + RADIX_BITS

    def perm(j, c):
        k = src_k[pl.ds(j * L, L)]
        i = src_i[pl.ds(j * L, L)]
        d = _digit(k, shift)
        cnt, last = plsc.scan_count(d)
        if nxt is not None:
            d2 = _digit(k, nshift)
            cnt2, last2 = plsc.scan_count(d2)
            plsc.addupdate_scatter(nxt, [d2], cnt2, mask=last2)
        pos = plsc.load_gather(hist, [d]) + cnt - 1
        plsc.store_scatter(dst_k, [pos], k)
        plsc.store_scatter(dst_i, [pos], i)
        plsc.addupdate_scatter(hist, [d], cnt, mask=last)
        return c

    lax.fori_loop(0, NV, perm, 0, unroll=4)


def _sc_body(keys_hbm, pw_hbm, idx_hbm, ka, kb, ia, ib, h0, h1, h2, w):
    nc = 2
    wid = lax.axis_index("s") * nc + lax.axis_index("c")
    pltpu.sync_copy(keys_hbm.at[wid], ka)

    lane = lax.iota(jnp.int32, L)

    def zero(j, c):
        h0[pl.ds(j * L, L)] = jnp.zeros((L,), jnp.int32)
        h1[pl.ds(j * L, L)] = jnp.zeros((L,), jnp.int32)
        h2[pl.ds(j * L, L)] = jnp.zeros((L,), jnp.int32)
        return c

    lax.fori_loop(0, NBINS // L, zero, 0, unroll=4)

    def init(j, c):
        ia[pl.ds(j * L, L)] = lane + j * L
        w[pl.ds(j * L, L)] = jnp.zeros((L,), jnp.float32)
        k = ka[pl.ds(j * L, L)]
        d = _digit(k, 0)
        cnt, last = plsc.scan_count(d)
        plsc.addupdate_scatter(h0, [d], cnt, mask=last)
        return c

    lax.fori_loop(0, NV, init, 0, unroll=4)

    _psum(h0)
    _perm_pass(0, ka, ia, kb, ib, h0, h1)
    _psum(h1)
    _perm_pass(RADIX_BITS, kb, ib, ka, ia, h1, h2)
    _psum(h2)
    _perm_pass(2 * RADIX_BITS, ka, ia, kb, ib, h2, None)

    vmax = jnp.max(_recover(kb[pl.ds(0, L)]))

    def tsum(j, acc):
        v = _recover(kb[pl.ds(j * L, L)])
        m = lane + j * L < G
        return acc + jnp.where(m, jnp.exp(v - vmax), 0.0)

    acc = lax.fori_loop(0, NT, tsum, jnp.zeros((L,), jnp.float32), unroll=2)
    tsumv = jnp.sum(acc)

    def tscat(j, c):
        v = _recover(kb[pl.ds(j * L, L)])
        i = ib[pl.ds(j * L, L)]
        m = lane + j * L < G
        plsc.store_scatter(w, [i], jnp.exp(v - vmax) / tsumv, mask=m)
        return c

    lax.fori_loop(0, NT, tscat, 0, unroll=2)

    base = NV - NT
    vmin = jnp.min(_recover(kb[pl.ds((NV - 1) * L, L)]))

    def bsum(j, acc):
        v = _recover(kb[pl.ds((base + j) * L, L)])
        m = lane + (base + j) * L >= N - G
        return acc + jnp.where(m, jnp.exp(vmin - v), 0.0)

    bacc = lax.fori_loop(0, NT, bsum, jnp.zeros((L,), jnp.float32), unroll=2)
    bsumv = jnp.sum(bacc)

    def bscat(j, c):
        v = _recover(kb[pl.ds((base + j) * L, L)])
        i = ib[pl.ds((base + j) * L, L)]
        m = lane + (base + j) * L >= N - G
        plsc.store_scatter(w, [i], -jnp.exp(vmin - v) / bsumv, mask=m)
        return c

    lax.fori_loop(0, NT, bscat, 0, unroll=2)

    pltpu.sync_copy(w, pw_hbm.at[wid])
    pltpu.sync_copy(ib, idx_hbm.at[wid])


@functools.cache
def _sc_sort():
    mesh = plsc.VectorSubcoreMesh(core_axis_name="c", subcore_axis_name="s")
    return pl.kernel(
        _sc_body,
        mesh=mesh,
        compiler_params=pltpu.CompilerParams(needs_layout_passes=False),
        out_type=[
            jax.ShapeDtypeStruct((B, N), jnp.float32),
            jax.ShapeDtypeStruct((B, N), jnp.int32),
        ],
        scratch_types=[
            pltpu.VMEM((N,), jnp.int32),
            pltpu.VMEM((N,), jnp.int32),
            pltpu.VMEM((N,), jnp.int32),
            pltpu.VMEM((N,), jnp.int32),
            pltpu.VMEM((NBINS,), jnp.int32),
            pltpu.VMEM((NBINS,), jnp.int32),
            pltpu.VMEM((NBINS,), jnp.int32),
            pltpu.VMEM((N,), jnp.float32),
        ],
    )


def kernel(s):
    keys = _maxkey(s)
    pw, idx = _sc_sort()(keys)
    return pw, idx
